# baseline (speedup 1.0000x reference)
import functools

import jax
import jax.numpy as jnp
from jax import lax
from jax.experimental import pallas as pl
from jax.experimental.pallas import tpu as pltpu
from jax.experimental.pallas import tpu_sc as plsc

_NBUF = 4


def _gather_kernel(n_per_w, chunk, n_chunks, idx_hbm, table_hbm, out_hbm,
                   ibuf, rbuf, si0, si1, si2, si3, sg0, sg1, sg2, sg3,
                   sw0, sw1, sw2, sw3):
    wid = lax.axis_index("s") * 2 + lax.axis_index("c")
    base = wid * n_per_w
    si = (si0, si1, si2, si3)
    sg = (sg0, sg1, sg2, sg3)
    sw = (sw0, sw1, sw2, sw3)

    def idx_start(c, b):
        pltpu.async_copy(idx_hbm.at[pl.ds(base + c * chunk, chunk)],
                         ibuf.at[b], si[b])

    def idx_wait(b):
        pltpu.make_async_copy(idx_hbm.at[pl.ds(0, chunk)], ibuf.at[b],
                              si[b]).wait()

    def gather_start(b):
        pltpu.async_copy(table_hbm.at[ibuf.at[b]], rbuf.at[b], sg[b])

    def gather_wait(b):
        pltpu.make_async_copy(table_hbm.at[ibuf.at[b]], rbuf.at[b],
                              sg[b]).wait()

    def wb_start(c, b):
        pltpu.async_copy(rbuf.at[b],
                         out_hbm.at[pl.ds(base + c * chunk, chunk)], sw[b])

    def wb_wait(b):
        pltpu.make_async_copy(rbuf.at[b], out_hbm.at[pl.ds(0, chunk)],
                              sw[b]).wait()

    for b in range(_NBUF):
        idx_start(b, b)
    for b in range(_NBUF):
        idx_wait(b)
        gather_start(b)

    @pl.loop(0, n_chunks, step=_NBUF)
    def _(g):
        for k in range(_NBUF):
            b = k
            bprev = (k - 1) % _NBUF
            c = g + k
            gather_wait(b)
            wb_start(c, b)

            @pl.when(c + _NBUF < n_chunks)
            def _():
                idx_start(c + _NBUF, b)

            fire_ok = c + _NBUF - 1 < n_chunks
            if k == 0:
                fire_cond = jnp.logical_and(g >= 1, fire_ok)
            else:
                fire_cond = fire_ok

            @pl.when(fire_cond)
            def _():
                wb_wait(bprev)
                idx_wait(bprev)
                gather_start(bprev)

    for b in range(_NBUF):
        wb_wait(b)


def kernel(tokens, table):
    b, l = tokens.shape
    v, d = table.shape
    n = b * l
    t0 = tokens.reshape(n).astype(jnp.int32) * 2
    idx = jnp.stack([t0, t0 + 1], axis=1).reshape(2 * n)
    table2 = jnp.pad(table, ((0, 0), (0, 128 - d))).reshape(2 * v, d)

    nw = 32
    n_per_w = 2 * n // nw
    chunk = 400
    n_chunks = n_per_w // chunk

    mesh = plsc.VectorSubcoreMesh(core_axis_name="c", subcore_axis_name="s")
    run = pl.kernel(
        functools.partial(_gather_kernel, n_per_w, chunk, n_chunks),
        mesh=mesh,
        out_type=jax.ShapeDtypeStruct((2 * n, d), jnp.float32),
        scratch_types=[
            pltpu.VMEM((_NBUF, chunk), jnp.int32),
            pltpu.VMEM((_NBUF, chunk, d), jnp.float32),
        ] + [pltpu.SemaphoreType.DMA] * (3 * _NBUF),
        compiler_params=pltpu.CompilerParams(use_tc_tiling_on_sc=False),
    )
    out = run(idx, table2)
    return out.reshape(b, l, 128)[:, :, :d]


# speedup vs baseline: 2.2576x; 2.2576x over previous
import functools

import jax
import jax.numpy as jnp
from jax import lax
from jax.experimental import pallas as pl
from jax.experimental.pallas import tpu as pltpu
from jax.experimental.pallas import tpu_sc as plsc

_NBUF = 4


def _gather_kernel(n_per_w, chunk, n_chunks, idx_hbm, table_hbm, out_hbm,
                   ibuf, rbuf, si0, si1, si2, si3, sg0, sg1, sg2, sg3,
                   sw0, sw1, sw2, sw3):
    wid = lax.axis_index("s") * 2 + lax.axis_index("c")
    base = wid * n_per_w
    si = (si0, si1, si2, si3)
    sg = (sg0, sg1, sg2, sg3)
    sw = (sw0, sw1, sw2, sw3)

    def idx_start(c, b):
        pltpu.async_copy(idx_hbm.at[pl.ds(base + c * chunk, chunk)],
                         ibuf.at[b], si[b])

    def idx_wait(b):
        pltpu.make_async_copy(idx_hbm.at[pl.ds(0, chunk)], ibuf.at[b],
                              si[b]).wait()

    def gather_start(b):
        pltpu.async_copy(table_hbm.at[ibuf.at[b]], rbuf.at[b], sg[b])

    def gather_wait(b):
        pltpu.make_async_copy(table_hbm.at[ibuf.at[b]], rbuf.at[b],
                              sg[b]).wait()

    def wb_start(c, b):
        pltpu.async_copy(
            rbuf.at[b],
            out_hbm.at[pl.ds(base + c * chunk, chunk), 0], sw[b])

    def wb_wait(b):
        pltpu.make_async_copy(rbuf.at[b], out_hbm.at[pl.ds(0, chunk), 0],
                              sw[b]).wait()

    for b in range(_NBUF):
        idx_start(b, b)
    for b in range(_NBUF):
        idx_wait(b)
        gather_start(b)

    @pl.loop(0, n_chunks, step=_NBUF)
    def _(g):
        for k in range(_NBUF):
            b = k
            bprev = (k - 1) % _NBUF
            c = g + k
            gather_wait(b)
            wb_start(c, b)

            @pl.when(c + _NBUF < n_chunks)
            def _():
                idx_start(c + _NBUF, b)

            fire_ok = c + _NBUF - 1 < n_chunks
            if k == 0:
                fire_cond = jnp.logical_and(g >= 1, fire_ok)
            else:
                fire_cond = fire_ok

            @pl.when(fire_cond)
            def _():
                wb_wait(bprev)
                idx_wait(bprev)
                gather_start(bprev)

    for b in range(_NBUF):
        wb_wait(b)


def kernel(tokens, table):
    b, l = tokens.shape
    v, d = table.shape
    n = b * l
    idx = tokens.reshape(n).astype(jnp.int32) * 2
    table2 = jnp.pad(table, ((0, 0), (0, 128 - d))).reshape(2 * v, d)

    nw = 32
    n_per_w = n // nw
    chunk = 400
    n_chunks = n_per_w // chunk

    mesh = plsc.VectorSubcoreMesh(core_axis_name="c", subcore_axis_name="s")
    run = pl.kernel(
        functools.partial(_gather_kernel, n_per_w, chunk, n_chunks),
        mesh=mesh,
        out_type=jax.ShapeDtypeStruct((n, 2, d), jnp.float32),
        scratch_types=[
            pltpu.VMEM((_NBUF, chunk), jnp.int32),
            pltpu.VMEM((_NBUF, chunk, d), jnp.float32),
        ] + [pltpu.SemaphoreType.DMA] * (3 * _NBUF),
        compiler_params=pltpu.CompilerParams(use_tc_tiling_on_sc=False),
    )
    out = run(idx, table2)
    return out.reshape(b, l, 128)[:, :, :d]
